# Initial kernel scaffold; baseline (speedup 1.0000x reference)
import jax
import jax.numpy as jnp
from jax.experimental import pallas as pl


def kernel(boxes, scores):
    raise NotImplementedError("write your pallas kernel here")


# speedup vs baseline: 51.9680x; 51.9680x over previous
import jax
import jax.numpy as jnp
from jax.experimental import pallas as pl
from jax.experimental.pallas import tpu as pltpu

_N = 5000
_B = 128
_NB = 40
_NPAD = _NB * _B
_T = 0.5


def _row2col(row, eye):
    return jnp.sum(row * eye, axis=1, keepdims=True)


def _col2row(col, eye):
    return jnp.sum(col * eye, axis=0, keepdims=True)


def _iou_tile(x1c, y1c, x2c, y2c, ac, x1r, y1r, x2r, y2r, ar):
    xx1 = jnp.maximum(x1c, x1r)
    yy1 = jnp.maximum(y1c, y1r)
    xx2 = jnp.minimum(x2c, x2r)
    yy2 = jnp.minimum(y2c, y2r)
    w = jnp.maximum(xx2 - xx1, 0.0)
    h = jnp.maximum(yy2 - yy1, 0.0)
    inter = w * h
    union = ac + ar - inter
    return inter / (union + 1e-9)


def _nms_kernel(x1_ref, y1_ref, x2_ref, y2_ref, s_ref, out_ref,
                rank_ref, sx1_ref, sy1_ref, sx2_ref, sy2_ref, sa_ref,
                keep_ref):
    f32 = jnp.float32
    i32 = jnp.int32
    sub = jax.lax.broadcasted_iota(i32, (_B, _B), 0)
    lane = jax.lax.broadcasted_iota(i32, (_B, _B), 1)
    eye = (sub == lane).astype(f32)
    ltmask = (sub < lane).astype(f32)
    sub_col_i = jax.lax.broadcasted_iota(i32, (_B, 1), 0)
    lane_row_i = jax.lax.broadcasted_iota(i32, (1, _B), 1)
    sub_col = sub_col_i.astype(f32)
    lane_row = lane_row_i.astype(f32)

    def rank_block(g, _):
        srow_g = s_ref[pl.ds(g, 1), :]
        scol_g = _row2col(srow_g, eye)
        gidx_col = g * _B + sub_col_i

        def rank_inner(n, acc):
            srow_n = s_ref[pl.ds(n, 1), :]
            idx_row = n * _B + lane_row_i
            gt = (srow_n > scol_g)
            eqlt = (srow_n == scol_g) & (idx_row < gidx_col)
            return acc + (gt | eqlt).astype(f32)

        accm = jax.lax.fori_loop(0, _NB, rank_inner, jnp.zeros((_B, _B), f32))
        rank_col = jnp.sum(accm, axis=1, keepdims=True)
        rank_ref[pl.ds(g, 1), :] = _col2row(rank_col, eye)
        return 0

    jax.lax.fori_loop(0, _NB, rank_block, 0)

    def perm_block(c, _):
        cf = c.astype(f32)
        targ_col = cf * _B + sub_col

        def perm_inner(n, accs):
            ax1, ay1, ax2, ay2 = accs
            rrow = rank_ref[pl.ds(n, 1), :]
            m = (rrow == targ_col).astype(f32)
            ax1 = ax1 + m * x1_ref[pl.ds(n, 1), :]
            ay1 = ay1 + m * y1_ref[pl.ds(n, 1), :]
            ax2 = ax2 + m * x2_ref[pl.ds(n, 1), :]
            ay2 = ay2 + m * y2_ref[pl.ds(n, 1), :]
            return ax1, ay1, ax2, ay2

        z = jnp.zeros((_B, _B), f32)
        ax1, ay1, ax2, ay2 = jax.lax.fori_loop(0, _NB, perm_inner,
                                               (z, z, z, z))
        x1c = jnp.sum(ax1, axis=1, keepdims=True)
        y1c = jnp.sum(ay1, axis=1, keepdims=True)
        x2c = jnp.sum(ax2, axis=1, keepdims=True)
        y2c = jnp.sum(ay2, axis=1, keepdims=True)
        x1r = _col2row(x1c, eye)
        y1r = _col2row(y1c, eye)
        x2r = _col2row(x2c, eye)
        y2r = _col2row(y2c, eye)
        sx1_ref[pl.ds(c, 1), :] = x1r
        sy1_ref[pl.ds(c, 1), :] = y1r
        sx2_ref[pl.ds(c, 1), :] = x2r
        sy2_ref[pl.ds(c, 1), :] = y2r
        sa_ref[pl.ds(c, 1), :] = (x2r - x1r) * (y2r - y1r)
        return 0

    jax.lax.fori_loop(0, _NB, perm_block, 0)

    keep_ref[...] = jnp.ones((_NB, _B), f32)

    def nms_block(b, _):
        x1r = sx1_ref[pl.ds(b, 1), :]
        y1r = sy1_ref[pl.ds(b, 1), :]
        x2r = sx2_ref[pl.ds(b, 1), :]
        y2r = sy2_ref[pl.ds(b, 1), :]
        ar = sa_ref[pl.ds(b, 1), :]
        x1c = _row2col(x1r, eye)
        y1c = _row2col(y1r, eye)
        x2c = _row2col(x2r, eye)
        y2c = _row2col(y2r, eye)
        ac = _row2col(ar, eye)

        iou_bb = _iou_tile(x1c, y1c, x2c, y2c, ac, x1r, y1r, x2r, y2r, ar)
        s_intra = (iou_bb > _T).astype(f32) * ltmask

        ext_row = keep_ref[pl.ds(b, 1), :]

        def relax_cond(carry):
            _, go = carry
            return go

        def relax_body(carry):
            krow, _ = carry
            kcol = _row2col(krow, eye)
            killed = jnp.max(s_intra * kcol, axis=0, keepdims=True)
            new = ext_row * (1.0 - killed)
            go = jnp.any(new != krow)
            return new, go

        krow, _ = jax.lax.while_loop(relax_cond, relax_body,
                                     (ext_row, jnp.bool_(True)))
        keep_ref[pl.ds(b, 1), :] = krow
        kcol = _row2col(krow, eye)

        def push(c, _):
            cx1 = sx1_ref[pl.ds(c, 1), :]
            cy1 = sy1_ref[pl.ds(c, 1), :]
            cx2 = sx2_ref[pl.ds(c, 1), :]
            cy2 = sy2_ref[pl.ds(c, 1), :]
            car = sa_ref[pl.ds(c, 1), :]
            iou_bc = _iou_tile(x1c, y1c, x2c, y2c, ac, cx1, cy1, cx2, cy2,
                               car)
            killed = jnp.max((iou_bc > _T).astype(f32) * kcol, axis=0,
                             keepdims=True)
            keep_ref[pl.ds(c, 1), :] = keep_ref[pl.ds(c, 1), :] * (1.0 - killed)
            return 0

        jax.lax.fori_loop(b + 1, _NB, push, 0)
        return 0

    jax.lax.fori_loop(0, _NB, nms_block, 0)

    def unsort_block(g, _):
        rrow = rank_ref[pl.ds(g, 1), :]
        rcol = _row2col(rrow, eye)

        def unsort_inner(n, acc):
            nf = n.astype(f32)
            pos_row = nf * _B + lane_row
            m = (rcol == pos_row).astype(f32)
            return acc + m * keep_ref[pl.ds(n, 1), :]

        accm = jax.lax.fori_loop(0, _NB, unsort_inner,
                                 jnp.zeros((_B, _B), f32))
        kcol = jnp.sum(accm, axis=1, keepdims=True)
        out_ref[pl.ds(g, 1), :] = s_ref[pl.ds(g, 1), :] * _col2row(kcol, eye)
        return 0

    jax.lax.fori_loop(0, _NB, unsort_block, 0)


def kernel(boxes, scores):
    pad = _NPAD - _N
    x1 = jnp.pad(boxes[:, 0], (0, pad)).reshape(_NB, _B)
    y1 = jnp.pad(boxes[:, 1], (0, pad)).reshape(_NB, _B)
    x2 = jnp.pad(boxes[:, 2], (0, pad)).reshape(_NB, _B)
    y2 = jnp.pad(boxes[:, 3], (0, pad)).reshape(_NB, _B)
    s = jnp.pad(scores, (0, pad), constant_values=-1.0).reshape(_NB, _B)

    out = pl.pallas_call(
        _nms_kernel,
        out_shape=jax.ShapeDtypeStruct((_NB, _B), jnp.float32),
        scratch_shapes=[pltpu.VMEM((_NB, _B), jnp.float32)] * 7,
    )(x1, y1, x2, y2, s)
    return out.reshape(-1)[:_N]


# speedup vs baseline: 66.8790x; 1.2869x over previous
import functools

import jax
import jax.numpy as jnp
from jax import lax
from jax.experimental import pallas as pl
from jax.experimental.pallas import tpu as pltpu
from jax.experimental.pallas import tpu_sc as plsc

_N = 5000
_B = 128
_NB = 40
_NPAD = _NB * _B
_T = 0.5

_NC = 2
_NS = 16
_NW = _NC * _NS
_L = 16
_CHUNK = _NPAD // _NW


def _row2col(row, eye):
    return jnp.sum(row * eye, axis=1, keepdims=True)


def _col2row(col, eye):
    return jnp.sum(col * eye, axis=0, keepdims=True)


def _iou_tile(x1c, y1c, x2c, y2c, ac, x1r, y1r, x2r, y2r, ar):
    xx1 = jnp.maximum(x1c, x1r)
    yy1 = jnp.maximum(y1c, y1r)
    xx2 = jnp.minimum(x2c, x2r)
    yy2 = jnp.minimum(y2c, y2r)
    w = jnp.maximum(xx2 - xx1, 0.0)
    h = jnp.maximum(yy2 - yy1, 0.0)
    inter = w * h
    union = ac + ar - inter
    return inter / (union + 1e-9)


def _rank_kernel(s_ref, rank_ref, inv_ref, rankf_ref):
    f32 = jnp.float32
    i32 = jnp.int32
    sub = jax.lax.broadcasted_iota(i32, (_B, _B), 0)
    lane = jax.lax.broadcasted_iota(i32, (_B, _B), 1)
    eye = (sub == lane).astype(f32)
    gtmask = (sub > lane).astype(f32)
    sub_col = jax.lax.broadcasted_iota(i32, (_B, 1), 0).astype(f32)
    lane_row = jax.lax.broadcasted_iota(i32, (1, _B), 1).astype(f32)

    def rank_block(g, _):
        srow_g = s_ref[pl.ds(g, 1), :]
        scol_g = _row2col(srow_g, eye)

        def rank_inner(n, acc):
            srow_n = s_ref[pl.ds(n, 1), :]
            gt = (srow_n > scol_g).astype(f32)
            eq = (srow_n == scol_g).astype(f32)
            w = jnp.where(n < g, 1.0, 0.0)
            return acc + gt + eq * w

        accm = jax.lax.fori_loop(0, _NB, rank_inner, jnp.zeros((_B, _B), f32))
        eq_gg = (srow_g == scol_g).astype(f32)
        accm = accm + eq_gg * gtmask
        rank_col = jnp.sum(accm, axis=1, keepdims=True)
        rank_row = _col2row(rank_col, eye)
        rankf_ref[pl.ds(g, 1), :] = rank_row
        rank_ref[pl.ds(g, 1), :] = rank_row.astype(i32)
        return 0

    jax.lax.fori_loop(0, _NB, rank_block, 0)

    def inv_block(c, _):
        cf = c.astype(jnp.float32)
        targ_col = cf * _B + sub_col

        def inv_inner(n, acc):
            nf = n.astype(jnp.float32)
            rrow = rankf_ref[pl.ds(n, 1), :]
            m = (rrow == targ_col).astype(jnp.float32)
            idx_row = nf * _B + lane_row
            return acc + m * idx_row

        accm = jax.lax.fori_loop(0, _NB, inv_inner,
                                 jnp.zeros((_B, _B), jnp.float32))
        inv_col = jnp.sum(accm, axis=1, keepdims=True)
        inv_ref[pl.ds(c, 1), :] = _col2row(inv_col, eye).astype(jnp.int32)
        return 0

    jax.lax.fori_loop(0, _NB, inv_block, 0)


def _nms_kernel(x1_ref, y1_ref, x2_ref, y2_ref, keep_ref):
    f32 = jnp.float32
    i32 = jnp.int32
    sub = jax.lax.broadcasted_iota(i32, (_B, _B), 0)
    lane = jax.lax.broadcasted_iota(i32, (_B, _B), 1)
    eye = (sub == lane).astype(f32)
    ltmask = (sub < lane).astype(f32)

    keep_ref[...] = jnp.ones((_NB, _B), f32)

    def nms_block(b, _):
        x1r = x1_ref[pl.ds(b, 1), :]
        y1r = y1_ref[pl.ds(b, 1), :]
        x2r = x2_ref[pl.ds(b, 1), :]
        y2r = y2_ref[pl.ds(b, 1), :]
        ar = (x2r - x1r) * (y2r - y1r)
        x1c = _row2col(x1r, eye)
        y1c = _row2col(y1r, eye)
        x2c = _row2col(x2r, eye)
        y2c = _row2col(y2r, eye)
        ac = _row2col(ar, eye)

        iou_bb = _iou_tile(x1c, y1c, x2c, y2c, ac, x1r, y1r, x2r, y2r, ar)
        s_intra = (iou_bb > _T).astype(f32) * ltmask

        ext_row = keep_ref[pl.ds(b, 1), :]

        def relax_cond(carry):
            _, go = carry
            return go

        def relax_body(carry):
            krow, _ = carry
            kcol = _row2col(krow, eye)
            killed = jnp.max(s_intra * kcol, axis=0, keepdims=True)
            new = ext_row * (1.0 - killed)
            go = jnp.any(new != krow)
            return new, go

        krow, _ = jax.lax.while_loop(relax_cond, relax_body,
                                     (ext_row, jnp.bool_(True)))
        keep_ref[pl.ds(b, 1), :] = krow
        kcol = _row2col(krow, eye)

        def push(c, _):
            cx1 = x1_ref[pl.ds(c, 1), :]
            cy1 = y1_ref[pl.ds(c, 1), :]
            cx2 = x2_ref[pl.ds(c, 1), :]
            cy2 = y2_ref[pl.ds(c, 1), :]
            car = (cx2 - cx1) * (cy2 - cy1)
            iou_bc = _iou_tile(x1c, y1c, x2c, y2c, ac, cx1, cy1, cx2, cy2,
                               car)
            killed = jnp.max((iou_bc > _T).astype(f32) * kcol, axis=0,
                             keepdims=True)
            keep_ref[pl.ds(c, 1), :] = keep_ref[pl.ds(c, 1), :] * (1.0 - killed)
            return 0

        jax.lax.fori_loop(b + 1, _NB, push, 0)
        return 0

    jax.lax.fori_loop(0, _NB, nms_block, 0)


@functools.lru_cache(maxsize=None)
def _sc_kernels():
    mesh = plsc.VectorSubcoreMesh(core_axis_name="c", subcore_axis_name="s",
                                  num_cores=_NC, num_subcores=_NS)

    @functools.partial(
        pl.kernel,
        out_type=[jax.ShapeDtypeStruct((_NPAD,), jnp.float32)] * 4,
        mesh=mesh,
        compiler_params=pltpu.CompilerParams(needs_layout_passes=False),
        scratch_types=[pltpu.VMEM((_NPAD,), jnp.float32)] * 4
        + [pltpu.VMEM((_CHUNK,), jnp.int32),
           pltpu.VMEM((_CHUNK,), jnp.float32)],
    )
    def sc_sort_gather(x1h, y1h, x2h, y2h, invh,
                       ox1, oy1, ox2, oy2,
                       x1v, y1v, x2v, y2v, idxv, outv):
        wid = lax.axis_index("s") * _NC + lax.axis_index("c")
        base = wid * _CHUNK
        pltpu.sync_copy(invh.at[pl.ds(base, _CHUNK)], idxv)
        pltpu.sync_copy(x1h, x1v)
        pltpu.sync_copy(y1h, y1v)
        pltpu.sync_copy(x2h, x2v)
        pltpu.sync_copy(y2h, y2v)
        for src, dst in ((x1v, ox1), (y1v, oy1), (x2v, ox2), (y2v, oy2)):
            for j in range(_CHUNK // _L):
                idx = idxv[pl.ds(j * _L, _L)]
                outv[pl.ds(j * _L, _L)] = plsc.load_gather(src, [idx])
            pltpu.sync_copy(outv, dst.at[pl.ds(base, _CHUNK)])

    @functools.partial(
        pl.kernel,
        out_type=jax.ShapeDtypeStruct((_NPAD,), jnp.float32),
        mesh=mesh,
        compiler_params=pltpu.CompilerParams(needs_layout_passes=False),
        scratch_types=[pltpu.VMEM((_NPAD,), jnp.float32),
                       pltpu.VMEM((_CHUNK,), jnp.int32),
                       pltpu.VMEM((_CHUNK,), jnp.float32),
                       pltpu.VMEM((_CHUNK,), jnp.float32)],
    )
    def sc_unsort_gather(keeph, rankh, sh, outh, keepv, rankv, sv, outv):
        wid = lax.axis_index("s") * _NC + lax.axis_index("c")
        base = wid * _CHUNK
        pltpu.sync_copy(keeph, keepv)
        pltpu.sync_copy(rankh.at[pl.ds(base, _CHUNK)], rankv)
        pltpu.sync_copy(sh.at[pl.ds(base, _CHUNK)], sv)
        for j in range(_CHUNK // _L):
            idx = rankv[pl.ds(j * _L, _L)]
            k = plsc.load_gather(keepv, [idx])
            outv[pl.ds(j * _L, _L)] = k * sv[pl.ds(j * _L, _L)]
        pltpu.sync_copy(outv, outh.at[pl.ds(base, _CHUNK)])

    return sc_sort_gather, sc_unsort_gather


def kernel(boxes, scores):
    pad = _NPAD - _N
    x1 = jnp.pad(boxes[:, 0], (0, pad))
    y1 = jnp.pad(boxes[:, 1], (0, pad))
    x2 = jnp.pad(boxes[:, 2], (0, pad))
    y2 = jnp.pad(boxes[:, 3], (0, pad))
    s = jnp.pad(scores, (0, pad), constant_values=-1.0)
    s2d = s.reshape(_NB, _B)

    rank2d, inv2d = pl.pallas_call(
        _rank_kernel,
        out_shape=[jax.ShapeDtypeStruct((_NB, _B), jnp.int32)] * 2,
        scratch_shapes=[pltpu.VMEM((_NB, _B), jnp.float32)],
    )(s2d)

    sc_sort_gather, sc_unsort_gather = _sc_kernels()
    sx1, sy1, sx2, sy2 = sc_sort_gather(x1, y1, x2, y2, inv2d.reshape(-1))

    keep2d = pl.pallas_call(
        _nms_kernel,
        out_shape=jax.ShapeDtypeStruct((_NB, _B), jnp.float32),
    )(sx1.reshape(_NB, _B), sy1.reshape(_NB, _B),
      sx2.reshape(_NB, _B), sy2.reshape(_NB, _B))

    out = sc_unsort_gather(keep2d.reshape(-1), rank2d.reshape(-1), s)
    return out[:_N]


# speedup vs baseline: 88.0990x; 1.3173x over previous
import functools

import jax
import jax.numpy as jnp
from jax import lax
from jax.experimental import pallas as pl
from jax.experimental.pallas import tpu as pltpu
from jax.experimental.pallas import tpu_sc as plsc

_N = 5000
_B = 128
_NB = 40
_NPAD = _NB * _B
_T = 0.5

_NC = 2
_NS = 16
_NW = _NC * _NS
_L = 16
_CHUNK = _NPAD // _NW


def _row2col(row, eye):
    return jnp.sum(row * eye, axis=1, keepdims=True)


def _col2row(col, eye):
    return jnp.sum(col * eye, axis=0, keepdims=True)


def _iou_tile(x1c, y1c, x2c, y2c, ac, x1r, y1r, x2r, y2r, ar):
    xx1 = jnp.maximum(x1c, x1r)
    yy1 = jnp.maximum(y1c, y1r)
    xx2 = jnp.minimum(x2c, x2r)
    yy2 = jnp.minimum(y2c, y2r)
    w = jnp.maximum(xx2 - xx1, 0.0)
    h = jnp.maximum(yy2 - yy1, 0.0)
    inter = w * h
    union = ac + ar - inter
    return inter / (union + 1e-9)


def _rank_kernel(s_ref, rank_ref):
    f32 = jnp.float32
    i32 = jnp.int32
    sub = jax.lax.broadcasted_iota(i32, (_B, _B), 0)
    lane = jax.lax.broadcasted_iota(i32, (_B, _B), 1)
    eye = (sub == lane).astype(f32)
    gtmask = (sub > lane).astype(f32)

    def rank_block(g, _):
        srow_g = s_ref[pl.ds(g, 1), :]
        scol_g = _row2col(srow_g, eye)

        def body_ge(n, acc):
            return acc + (s_ref[pl.ds(n, 1), :] >= scol_g).astype(f32)

        def body_gt(n, acc):
            return acc + (s_ref[pl.ds(n, 1), :] > scol_g).astype(f32)

        accm = jax.lax.fori_loop(0, g, body_ge, jnp.zeros((_B, _B), f32))
        accm = jax.lax.fori_loop(g + 1, _NB, body_gt, accm)
        gt_gg = (srow_g > scol_g).astype(f32)
        eq_gg = (srow_g == scol_g).astype(f32)
        accm = accm + gt_gg + eq_gg * gtmask
        rank_col = jnp.sum(accm, axis=1, keepdims=True)
        rank_ref[pl.ds(g, 1), :] = _col2row(rank_col, eye).astype(i32)
        return 0

    jax.lax.fori_loop(0, _NB, rank_block, 0)


def _nms_kernel(x1_ref, y1_ref, x2_ref, y2_ref, keep_ref):
    f32 = jnp.float32
    i32 = jnp.int32
    sub = jax.lax.broadcasted_iota(i32, (_B, _B), 0)
    lane = jax.lax.broadcasted_iota(i32, (_B, _B), 1)
    eye = (sub == lane).astype(f32)
    ltmask = (sub < lane).astype(f32)

    keep_ref[...] = jnp.ones((_NB, _B), f32)

    def nms_block(b, _):
        x1r = x1_ref[pl.ds(b, 1), :]
        y1r = y1_ref[pl.ds(b, 1), :]
        x2r = x2_ref[pl.ds(b, 1), :]
        y2r = y2_ref[pl.ds(b, 1), :]
        ar = (x2r - x1r) * (y2r - y1r)
        x1c = _row2col(x1r, eye)
        y1c = _row2col(y1r, eye)
        x2c = _row2col(x2r, eye)
        y2c = _row2col(y2r, eye)
        ac = _row2col(ar, eye)

        iou_bb = _iou_tile(x1c, y1c, x2c, y2c, ac, x1r, y1r, x2r, y2r, ar)
        s_intra = (iou_bb > _T).astype(f32) * ltmask

        ext_row = keep_ref[pl.ds(b, 1), :]

        def relax_cond(carry):
            _, go = carry
            return go

        def relax_body(carry):
            krow, _ = carry
            kcol = _row2col(krow, eye)
            killed = jnp.max(s_intra * kcol, axis=0, keepdims=True)
            new = ext_row * (1.0 - killed)
            go = jnp.any(new != krow)
            return new, go

        krow, _ = jax.lax.while_loop(relax_cond, relax_body,
                                     (ext_row, jnp.bool_(True)))
        keep_ref[pl.ds(b, 1), :] = krow
        kcol = _row2col(krow, eye)

        def push(c, _):
            cx1 = x1_ref[pl.ds(c, 1), :]
            cy1 = y1_ref[pl.ds(c, 1), :]
            cx2 = x2_ref[pl.ds(c, 1), :]
            cy2 = y2_ref[pl.ds(c, 1), :]
            car = (cx2 - cx1) * (cy2 - cy1)
            iou_bc = _iou_tile(x1c, y1c, x2c, y2c, ac, cx1, cy1, cx2, cy2,
                               car)
            killed = jnp.max((iou_bc > _T).astype(f32) * kcol, axis=0,
                             keepdims=True)
            keep_ref[pl.ds(c, 1), :] = keep_ref[pl.ds(c, 1), :] * (1.0 - killed)
            return 0

        jax.lax.fori_loop(b + 1, _NB, push, 0)
        return 0

    jax.lax.fori_loop(0, _NB, nms_block, 0)


@functools.lru_cache(maxsize=None)
def _sc_kernels():
    mesh = plsc.VectorSubcoreMesh(core_axis_name="c", subcore_axis_name="s",
                                  num_cores=_NC, num_subcores=_NS)

    @functools.partial(
        pl.kernel,
        out_type=[jax.ShapeDtypeStruct((_NPAD,), jnp.float32)] * 4,
        mesh=mesh,
        compiler_params=pltpu.CompilerParams(needs_layout_passes=False),
        scratch_types=[pltpu.VMEM((_NPAD,), jnp.float32)] * 4
        + [pltpu.VMEM((_NPAD,), jnp.int32),
           pltpu.VMEM((_NPAD,), jnp.int32),
           pltpu.VMEM((_CHUNK,), jnp.float32)],
    )
    def sc_sort_gather(x1h, y1h, x2h, y2h, rankh,
                       ox1, oy1, ox2, oy2,
                       x1v, y1v, x2v, y2v, rankv, invv, outv):
        wid = lax.axis_index("s") * _NC + lax.axis_index("c")
        base = wid * _CHUNK
        pltpu.sync_copy(rankh, rankv)
        pltpu.sync_copy(x1h, x1v)
        pltpu.sync_copy(y1h, y1v)
        pltpu.sync_copy(x2h, x2v)
        pltpu.sync_copy(y2h, y2v)

        lane = lax.iota(jnp.int32, _L)

        def inv_body(j, _):
            idx = rankv[pl.ds(j * _L, _L)]
            plsc.store_scatter(invv, [idx], j * _L + lane)
            return 0

        lax.fori_loop(0, _NPAD // _L, inv_body, 0)

        for src, dst in ((x1v, ox1), (y1v, oy1), (x2v, ox2), (y2v, oy2)):
            for j in range(_CHUNK // _L):
                idx = invv[pl.ds(base + j * _L, _L)]
                outv[pl.ds(j * _L, _L)] = plsc.load_gather(src, [idx])
            pltpu.sync_copy(outv, dst.at[pl.ds(base, _CHUNK)])

    @functools.partial(
        pl.kernel,
        out_type=jax.ShapeDtypeStruct((_NPAD,), jnp.float32),
        mesh=mesh,
        compiler_params=pltpu.CompilerParams(needs_layout_passes=False),
        scratch_types=[pltpu.VMEM((_NPAD,), jnp.float32),
                       pltpu.VMEM((_CHUNK,), jnp.int32),
                       pltpu.VMEM((_CHUNK,), jnp.float32),
                       pltpu.VMEM((_CHUNK,), jnp.float32)],
    )
    def sc_unsort_gather(keeph, rankh, sh, outh, keepv, rankv, sv, outv):
        wid = lax.axis_index("s") * _NC + lax.axis_index("c")
        base = wid * _CHUNK
        pltpu.sync_copy(keeph, keepv)
        pltpu.sync_copy(rankh.at[pl.ds(base, _CHUNK)], rankv)
        pltpu.sync_copy(sh.at[pl.ds(base, _CHUNK)], sv)
        for j in range(_CHUNK // _L):
            idx = rankv[pl.ds(j * _L, _L)]
            k = plsc.load_gather(keepv, [idx])
            outv[pl.ds(j * _L, _L)] = k * sv[pl.ds(j * _L, _L)]
        pltpu.sync_copy(outv, outh.at[pl.ds(base, _CHUNK)])

    return sc_sort_gather, sc_unsort_gather


def kernel(boxes, scores):
    pad = _NPAD - _N
    x1 = jnp.pad(boxes[:, 0], (0, pad))
    y1 = jnp.pad(boxes[:, 1], (0, pad))
    x2 = jnp.pad(boxes[:, 2], (0, pad))
    y2 = jnp.pad(boxes[:, 3], (0, pad))
    s = jnp.pad(scores, (0, pad), constant_values=-1.0)
    s2d = s.reshape(_NB, _B)

    rank2d = pl.pallas_call(
        _rank_kernel,
        out_shape=jax.ShapeDtypeStruct((_NB, _B), jnp.int32),
    )(s2d)

    sc_sort_gather, sc_unsort_gather = _sc_kernels()
    sx1, sy1, sx2, sy2 = sc_sort_gather(x1, y1, x2, y2, rank2d.reshape(-1))

    keep2d = pl.pallas_call(
        _nms_kernel,
        out_shape=jax.ShapeDtypeStruct((_NB, _B), jnp.float32),
    )(sx1.reshape(_NB, _B), sy1.reshape(_NB, _B),
      sx2.reshape(_NB, _B), sy2.reshape(_NB, _B))

    out = sc_unsort_gather(keep2d.reshape(-1), rank2d.reshape(-1), s)
    return out[:_N]
